# Initial kernel scaffold; baseline (speedup 1.0000x reference)
import jax
import jax.numpy as jnp
from jax.experimental import pallas as pl


def kernel(input, memory_bank, W_in, W_out):
    raise NotImplementedError("write your pallas kernel here")


# speedup vs baseline: 1.4093x; 1.4093x over previous
import jax
import jax.numpy as jnp
from jax.experimental import pallas as pl

B, T, S, D = 32, 8, 2048, 1024


def _fused_attn_kernel(x_ref, mb_ref, win_ref, wout_ref,
                       attn_ref, alpha_ref, scores_ref):
    x = x_ref[0]
    mb = mb_ref[0]
    h = jax.lax.dot_general(x, win_ref[...], (((1,), (1,)), ((), ())),
                            preferred_element_type=jnp.float32)
    s = jax.lax.dot_general(h, mb, (((1,), (1,)), ((), ())),
                            preferred_element_type=jnp.float32)
    scores_ref[0] = s
    m = jnp.max(s, axis=-1, keepdims=True)
    e = jnp.exp(s - m)
    denom = jnp.sum(e, axis=-1, keepdims=True)
    a = e / denom
    alpha_ref[0] = a
    c = jnp.dot(a, mb, preferred_element_type=jnp.float32)
    w_c = wout_ref[:, :D]
    w_x = wout_ref[:, D:]
    out = (jax.lax.dot_general(c, w_c, (((1,), (1,)), ((), ())),
                               preferred_element_type=jnp.float32)
           + jax.lax.dot_general(x, w_x, (((1,), (1,)), ((), ())),
                                 preferred_element_type=jnp.float32))
    attn_ref[0] = jnp.tanh(out)


def kernel(input, memory_bank, W_in, W_out):
    grid = (B,)
    out_shapes = (
        jax.ShapeDtypeStruct((B, T, D), jnp.float32),
        jax.ShapeDtypeStruct((B, T, S), jnp.float32),
        jax.ShapeDtypeStruct((B, T, S), jnp.float32),
    )
    return pl.pallas_call(
        _fused_attn_kernel,
        grid=grid,
        in_specs=[
            pl.BlockSpec((1, T, D), lambda b: (b, 0, 0)),
            pl.BlockSpec((1, S, D), lambda b: (b, 0, 0)),
            pl.BlockSpec((D, D), lambda b: (0, 0)),
            pl.BlockSpec((D, 2 * D), lambda b: (0, 0)),
        ],
        out_specs=(
            pl.BlockSpec((1, T, D), lambda b: (b, 0, 0)),
            pl.BlockSpec((1, T, S), lambda b: (b, 0, 0)),
            pl.BlockSpec((1, T, S), lambda b: (b, 0, 0)),
        ),
        out_shape=out_shapes,
    )(input, memory_bank, W_in, W_out)


# speedup vs baseline: 1.6938x; 1.2019x over previous
import jax
import jax.numpy as jnp
from jax.experimental import pallas as pl

B, T, S, D = 32, 8, 2048, 1024


def _proj_in_kernel(x_ref, win_ref, h_ref):
    h_ref[...] = jax.lax.dot_general(
        x_ref[...], win_ref[...], (((1,), (1,)), ((), ())),
        preferred_element_type=jnp.float32)


def _attn_core_kernel(h_ref, mb_ref, scores_ref, alpha_ref, c_ref):
    h = h_ref[0]
    mb = mb_ref[0]
    s = jax.lax.dot_general(h, mb, (((1,), (1,)), ((), ())),
                            preferred_element_type=jnp.float32)
    scores_ref[0] = s
    m = jnp.max(s, axis=-1, keepdims=True)
    e = jnp.exp(s - m)
    denom = jnp.sum(e, axis=-1, keepdims=True)
    a = e / denom
    alpha_ref[0] = a
    c_ref[0] = jnp.dot(a, mb, preferred_element_type=jnp.float32)


def _proj_out_kernel(c_ref, x_ref, wout_ref, attn_ref):
    w_c = wout_ref[:, :D]
    w_x = wout_ref[:, D:]
    out = (jax.lax.dot_general(c_ref[...], w_c, (((1,), (1,)), ((), ())),
                               preferred_element_type=jnp.float32)
           + jax.lax.dot_general(x_ref[...], w_x, (((1,), (1,)), ((), ())),
                                 preferred_element_type=jnp.float32))
    attn_ref[...] = jnp.tanh(out)


def kernel(input, memory_bank, W_in, W_out):
    x2d = input.reshape(B * T, D)

    h2d = pl.pallas_call(
        _proj_in_kernel,
        out_shape=jax.ShapeDtypeStruct((B * T, D), jnp.float32),
    )(x2d, W_in)

    h3d = h2d.reshape(B, T, D)
    scores, alpha, c = pl.pallas_call(
        _attn_core_kernel,
        grid=(B,),
        in_specs=[
            pl.BlockSpec((1, T, D), lambda b: (b, 0, 0)),
            pl.BlockSpec((1, S, D), lambda b: (b, 0, 0)),
        ],
        out_specs=(
            pl.BlockSpec((1, T, S), lambda b: (b, 0, 0)),
            pl.BlockSpec((1, T, S), lambda b: (b, 0, 0)),
            pl.BlockSpec((1, T, D), lambda b: (b, 0, 0)),
        ),
        out_shape=(
            jax.ShapeDtypeStruct((B, T, S), jnp.float32),
            jax.ShapeDtypeStruct((B, T, S), jnp.float32),
            jax.ShapeDtypeStruct((B, T, D), jnp.float32),
        ),
    )(h3d, memory_bank)

    attn2d = pl.pallas_call(
        _proj_out_kernel,
        out_shape=jax.ShapeDtypeStruct((B * T, D), jnp.float32),
    )(c.reshape(B * T, D), x2d, W_out)

    return (attn2d.reshape(B, T, D), alpha, scores)


# speedup vs baseline: 1.7706x; 1.0453x over previous
import jax
import jax.numpy as jnp
from jax.experimental import pallas as pl
from jax.experimental.pallas import tpu as pltpu

B, T, S, D = 32, 8, 2048, 1024


def _fused_kernel(x_ref, mb_ref, win_ref, wout_ref,
                  scores_ref, alpha_ref, attn_ref,
                  h_scr, c_scr):
    b = pl.program_id(0)

    @pl.when(b == 0)
    def _prologue():
        h_scr[...] = jax.lax.dot_general(
            x_ref[...], win_ref[...], (((1,), (1,)), ((), ())),
            preferred_element_type=jnp.float32)

    h = h_scr[pl.ds(b * T, T), :]
    mb = mb_ref[0]
    s = jax.lax.dot_general(h, mb, (((1,), (1,)), ((), ())),
                            preferred_element_type=jnp.float32)
    scores_ref[0] = s
    m = jnp.max(s, axis=-1, keepdims=True)
    e = jnp.exp(s - m)
    denom = jnp.sum(e, axis=-1, keepdims=True)
    a = e / denom
    alpha_ref[0] = a
    c_scr[pl.ds(b * T, T), :] = jnp.dot(a, mb,
                                        preferred_element_type=jnp.float32)

    @pl.when(b == B - 1)
    def _epilogue():
        w_c = wout_ref[:, :D]
        w_x = wout_ref[:, D:]
        out = (jax.lax.dot_general(c_scr[...], w_c, (((1,), (1,)), ((), ())),
                                   preferred_element_type=jnp.float32)
               + jax.lax.dot_general(x_ref[...], w_x, (((1,), (1,)), ((), ())),
                                     preferred_element_type=jnp.float32))
        attn_ref[...] = jnp.tanh(out)


def kernel(input, memory_bank, W_in, W_out):
    x2d = input.reshape(B * T, D)

    scores, alpha, attn2d = pl.pallas_call(
        _fused_kernel,
        grid=(B,),
        in_specs=[
            pl.BlockSpec((B * T, D), lambda b: (0, 0)),
            pl.BlockSpec((1, S, D), lambda b: (b, 0, 0)),
            pl.BlockSpec((D, D), lambda b: (0, 0)),
            pl.BlockSpec((D, 2 * D), lambda b: (0, 0)),
        ],
        out_specs=(
            pl.BlockSpec((1, T, S), lambda b: (b, 0, 0)),
            pl.BlockSpec((1, T, S), lambda b: (b, 0, 0)),
            pl.BlockSpec((B * T, D), lambda b: (0, 0)),
        ),
        out_shape=(
            jax.ShapeDtypeStruct((B, T, S), jnp.float32),
            jax.ShapeDtypeStruct((B, T, S), jnp.float32),
            jax.ShapeDtypeStruct((B * T, D), jnp.float32),
        ),
        scratch_shapes=[
            pltpu.VMEM((B * T, D), jnp.float32),
            pltpu.VMEM((B * T, D), jnp.float32),
        ],
    )(x2d, memory_bank, W_in, W_out)

    return (attn2d.reshape(B, T, D), alpha, scores)


# speedup vs baseline: 1.7818x; 1.0063x over previous
import jax
import jax.numpy as jnp
from jax.experimental import pallas as pl
from jax.experimental.pallas import tpu as pltpu

B, T, S, D = 32, 8, 2048, 1024


def _fused_kernel(x_ref, mb_ref, win_ref, wout_hbm,
                  scores_ref, alpha_ref, attn_ref,
                  h_scr, c_scr, wout_scr, wout_sem):
    b = pl.program_id(0)

    @pl.when(b == 0)
    def _prologue():
        pltpu.make_async_copy(wout_hbm, wout_scr, wout_sem).start()
        h_scr[...] = jax.lax.dot_general(
            x_ref[...], win_ref[...], (((1,), (1,)), ((), ())),
            preferred_element_type=jnp.float32)

    h = h_scr[pl.ds(b * T, T), :]
    mb = mb_ref[0]
    s = jax.lax.dot_general(h, mb, (((1,), (1,)), ((), ())),
                            preferred_element_type=jnp.float32)
    scores_ref[0] = s
    m = jnp.max(s, axis=-1, keepdims=True)
    e = jnp.exp(s - m)
    denom = jnp.sum(e, axis=-1, keepdims=True)
    a = e / denom
    alpha_ref[0] = a
    c_scr[pl.ds(b * T, T), :] = jnp.dot(a, mb,
                                        preferred_element_type=jnp.float32)

    @pl.when(b == B - 1)
    def _epilogue():
        pltpu.make_async_copy(wout_hbm, wout_scr, wout_sem).wait()
        w_c = wout_scr[:, :D]
        w_x = wout_scr[:, D:]
        out = (jax.lax.dot_general(c_scr[...], w_c, (((1,), (1,)), ((), ())),
                                   preferred_element_type=jnp.float32)
               + jax.lax.dot_general(x_ref[...], w_x, (((1,), (1,)), ((), ())),
                                     preferred_element_type=jnp.float32))
        attn_ref[...] = jnp.tanh(out)


def kernel(input, memory_bank, W_in, W_out):
    x2d = input.reshape(B * T, D)

    scores, alpha, attn2d = pl.pallas_call(
        _fused_kernel,
        grid=(B,),
        in_specs=[
            pl.BlockSpec((B * T, D), lambda b: (0, 0)),
            pl.BlockSpec((1, S, D), lambda b: (b, 0, 0)),
            pl.BlockSpec((D, D), lambda b: (0, 0)),
            pl.BlockSpec(memory_space=pl.ANY),
        ],
        out_specs=(
            pl.BlockSpec((1, T, S), lambda b: (b, 0, 0)),
            pl.BlockSpec((1, T, S), lambda b: (b, 0, 0)),
            pl.BlockSpec((B * T, D), lambda b: (0, 0)),
        ),
        out_shape=(
            jax.ShapeDtypeStruct((B, T, S), jnp.float32),
            jax.ShapeDtypeStruct((B, T, S), jnp.float32),
            jax.ShapeDtypeStruct((B * T, D), jnp.float32),
        ),
        scratch_shapes=[
            pltpu.VMEM((B * T, D), jnp.float32),
            pltpu.VMEM((B * T, D), jnp.float32),
            pltpu.VMEM((D, 2 * D), jnp.float32),
            pltpu.SemaphoreType.DMA,
        ],
    )(x2d, memory_bank, W_in, W_out)

    return (attn2d.reshape(B, T, D), alpha, scores)
